# baseline (speedup 1.0000x reference)
import dataclasses
import functools

import jax
import jax.numpy as jnp
from jax import lax
from jax.experimental import pallas as pl
from jax.experimental.pallas import tpu as pltpu
from jax.experimental.pallas import tpu_sc as plsc

_B, _S, _D = 4096, 200, 64
_V = 90
_VP = 96
_NC, _NS = 2, 16
_NW = _NC * _NS
_DG, _DS = _D // 8, 8
_BG, _BL = _B // 128, 128
_SG = _S // 8
_BGC = 2
_NBC = _BG // _BGC
_L = 16

_SG_TC = 12
_S_TC = _SG_TC * 8
_S_SC = _S - _S_TC
_SPW = _S_SC * _NBC // _NW

_TC_BGC = 8
_TC_NBC = _BG // _TC_BGC

_CP = pltpu.CompilerParams(use_tc_tiling_on_sc=False)
if "needs_layout_passes" in pltpu.CompilerParams.__dataclass_fields__:
    _CP = dataclasses.replace(_CP, needs_layout_passes=False)


def _sc_embed_add(x5, m3, t_flat):
    mesh = plsc.VectorSubcoreMesh(core_axis_name="core",
                                  subcore_axis_name="subcore")

    @functools.partial(
        pl.kernel,
        out_type=jax.ShapeDtypeStruct((_S_SC, _DG, _BG, _DS, _BL),
                                      jnp.float32),
        mesh=mesh,
        scratch_types=(
            [pltpu.VMEM((3, _DG, _BGC, _DS, _BL), jnp.float32),
             pltpu.VMEM((3, _BGC, _BL), jnp.int32),
             pltpu.VMEM((_D * _V,), jnp.float32)]
            + [pltpu.SemaphoreType.DMA] * 9
        ),
        compiler_params=_CP,
    )
    def k(x_hbm, i_hbm, t_hbm, o_hbm, io, ib, tv, *sems):
        sx = sems[0:3]
        si = sems[3:6]
        so = sems[6:9]
        wid = lax.axis_index("core") * _NS + lax.axis_index("subcore")
        base = wid * _SPW

        pltpu.sync_copy(t_hbm, tv)

        def addr(g):
            t = base + g
            return t // _NBC, t % _NBC

        def start_in(g, slot):
            s, c = addr(g)
            sx_ = s + _S_TC
            pltpu.async_copy(x_hbm.at[sx_, :, pl.ds(c * _BGC, _BGC)],
                             io.at[slot], sx[slot])
            pltpu.async_copy(
                i_hbm.at[sx_ // 8, pl.ds(c * _BGC, _BGC),
                         pl.ds((sx_ % 8) * _BL, _BL)],
                ib.at[slot], si[slot])

        def wait_in(slot):
            pltpu.make_async_copy(x_hbm.at[0, :, pl.ds(0, _BGC)],
                                  io.at[slot], sx[slot]).wait()
            pltpu.make_async_copy(
                i_hbm.at[0, pl.ds(0, _BGC), pl.ds(0, _BL)],
                ib.at[slot], si[slot]).wait()

        def start_out(g, slot):
            s, c = addr(g)
            pltpu.async_copy(io.at[slot],
                             o_hbm.at[s, :, pl.ds(c * _BGC, _BGC)], so[slot])

        def wait_out(slot):
            pltpu.make_async_copy(io.at[slot],
                                  o_hbm.at[0, :, pl.ds(0, _BGC)],
                                  so[slot]).wait()

        nvec = _BGC * _BL // _L
        grp = 8

        def jsl(j):
            return (j // (_BL // _L), pl.ds((j % (_BL // _L)) * _L, _L))

        def compute(slot):
            def dgloop(dg, carry):
                new = []
                for g0 in range(0, nvec, grp):
                    js = range(g0, g0 + grp)
                    for ds in range(_DS):
                        embs = [plsc.load_gather(tv, [carry[j] + ds * _V])
                                for j in js]
                        for j, ev in zip(js, embs):
                            bg, sl = jsl(j)
                            plsc.addupdate(io.at[slot, dg, bg, ds, sl], ev)
                    new.extend(carry[j] + _DS * _V for j in js)
                return tuple(new)

            init = []
            for j in range(nvec):
                bg, sl = jsl(j)
                init.append(ib.at[slot, bg, sl][...])
            lax.fori_loop(0, _DG, dgloop, tuple(init))

        start_in(0, 0)
        start_in(1, 1)
        wait_in(0)
        compute(0)
        start_out(0, 0)
        start_in(2, 2)

        @pl.loop(0, (_SPW - 1) // 3)
        def _(i):
            for sub in range(3):
                t = 3 * i + 1 + sub
                slot = (1 + sub) % 3
                nslot = sub
                wait_in(slot)
                compute(slot)
                start_out(t, slot)

                @pl.when(t + 2 < _SPW)
                def _():
                    wait_out(nslot)
                    start_in(t + 2, nslot)

        for slot in range(3):
            wait_out(slot)

    return k(x5, m3, t_flat)


def _tc_body(m_ref, x_ref, t_ref, o_ref):
    tt = t_ref[...]
    for si in range(8):
        for bg in range(_TC_BGC):
            m = m_ref[0, bg, si, :]
            iot = jax.lax.broadcasted_iota(jnp.int32, (_VP, _BL), 0)
            onehot = (iot == m[None, :]).astype(jnp.float32)
            p = jax.lax.dot_general(
                tt, onehot, (((1,), (0,)), ((), ())),
                preferred_element_type=jnp.float32)
            xv = x_ref[si, :, bg, :, :].reshape(_D, _BL)
            o_ref[si, :, bg, :, :] = (xv + p).reshape(_DG, _DS, _BL)


def _tc_embed_add(x5, m5, t_pad):
    return pl.pallas_call(
        _tc_body,
        grid=(_SG_TC, _TC_NBC),
        in_specs=[
            pl.BlockSpec((1, _TC_BGC, 8, _BL), lambda i, j: (i, j, 0, 0)),
            pl.BlockSpec((8, _DG, _TC_BGC, _DS, _BL),
                         lambda i, j: (i, 0, j, 0, 0)),
            pl.BlockSpec((_D, _VP), lambda i, j: (0, 0)),
        ],
        out_specs=pl.BlockSpec((8, _DG, _TC_BGC, _DS, _BL),
                               lambda i, j: (i, 0, j, 0, 0)),
        out_shape=jax.ShapeDtypeStruct((_S_TC, _DG, _BG, _DS, _BL),
                                       jnp.float32),
    )(m5, x5, t_pad)


@jax.jit
def kernel(x, minutes, table):
    x5 = (x.transpose(1, 2, 0)
           .reshape(_S, _DG, _DS, _BG, _BL)
           .transpose(0, 1, 3, 2, 4))
    m5 = (minutes.astype(jnp.int32).transpose(1, 0)
          .reshape(_SG, 8, _BG, _BL)
          .transpose(0, 2, 1, 3))
    m3 = m5.reshape(_SG, _BG, 8 * _BL)
    t_flat = table.T.reshape(_D * _V)
    t_pad = jnp.pad(table.T, ((0, 0), (0, _VP - _V)))
    o_tc = _tc_embed_add(x5, m5, t_pad)
    o_sc = _sc_embed_add(x5, m3, t_flat)
    o5 = jnp.concatenate([o_tc, o_sc], axis=0)
    return (o5.transpose(0, 1, 3, 2, 4)
              .reshape(_S, _D, _B)
              .transpose(2, 0, 1))


# speedup vs baseline: 1.5561x; 1.5561x over previous
import dataclasses
import functools

import jax
import jax.numpy as jnp
from jax import lax
from jax.experimental import pallas as pl
from jax.experimental.pallas import tpu as pltpu
from jax.experimental.pallas import tpu_sc as plsc

_B, _S, _D = 4096, 200, 64
_V = 90
_NC, _NS = 2, 16
_NW = _NC * _NS
_DG, _DS = _D // 8, 8
_BG, _BL = _B // 128, 128
_SG = _S // 8
_BGC = 2
_NBC = _BG // _BGC
_STEPS = _S * _NBC
_SPW = _STEPS // _NW
_L = 16

_CP = pltpu.CompilerParams(use_tc_tiling_on_sc=False)
if "needs_layout_passes" in pltpu.CompilerParams.__dataclass_fields__:
    _CP = dataclasses.replace(_CP, needs_layout_passes=False)


def _sc_embed_add(x5, m3, t_flat):
    mesh = plsc.VectorSubcoreMesh(core_axis_name="core",
                                  subcore_axis_name="subcore")

    @functools.partial(
        pl.kernel,
        out_type=jax.ShapeDtypeStruct((_S, _DG, _BG, _DS, _BL), jnp.float32),
        mesh=mesh,
        scratch_types=(
            [pltpu.VMEM((3, _DG, _BGC, _DS, _BL), jnp.float32),
             pltpu.VMEM((3, _BGC, _BL), jnp.int32),
             pltpu.VMEM((_D * _V,), jnp.float32)]
            + [pltpu.SemaphoreType.DMA] * 9
        ),
        compiler_params=_CP,
    )
    def k(x_hbm, i_hbm, t_hbm, o_hbm, io, ib, tv, *sems):
        sx = sems[0:3]
        si = sems[3:6]
        so = sems[6:9]
        wid = lax.axis_index("core") * _NS + lax.axis_index("subcore")
        base = wid * _SPW

        pltpu.sync_copy(t_hbm, tv)

        def addr(g):
            t = base + g
            s = t // _NBC
            c = t % _NBC
            return s, c

        def start_in(g, slot):
            s, c = addr(g)
            pltpu.async_copy(x_hbm.at[s, :, pl.ds(c * _BGC, _BGC)],
                             io.at[slot], sx[slot])
            pltpu.async_copy(
                i_hbm.at[s // 8, pl.ds(c * _BGC, _BGC),
                         pl.ds((s % 8) * _BL, _BL)],
                ib.at[slot], si[slot])

        def wait_in(slot):
            pltpu.make_async_copy(x_hbm.at[0, :, pl.ds(0, _BGC)],
                                  io.at[slot], sx[slot]).wait()
            pltpu.make_async_copy(
                i_hbm.at[0, pl.ds(0, _BGC), pl.ds(0, _BL)],
                ib.at[slot], si[slot]).wait()

        def start_out(g, slot):
            s, c = addr(g)
            pltpu.async_copy(io.at[slot],
                             o_hbm.at[s, :, pl.ds(c * _BGC, _BGC)], so[slot])

        def wait_out(slot):
            pltpu.make_async_copy(io.at[slot],
                                  o_hbm.at[0, :, pl.ds(0, _BGC)],
                                  so[slot]).wait()

        nvec = _BGC * _BL // _L
        grp = 8

        def jsl(j):
            return (j // (_BL // _L), pl.ds((j % (_BL // _L)) * _L, _L))

        def compute(slot):
            def dgloop(dg, carry):
                new = []
                for g0 in range(0, nvec, grp):
                    js = range(g0, g0 + grp)
                    for ds in range(_DS):
                        embs = [plsc.load_gather(tv, [carry[j] + ds * _V])
                                for j in js]
                        for j, ev in zip(js, embs):
                            bg, sl = jsl(j)
                            plsc.addupdate(io.at[slot, dg, bg, ds, sl], ev)
                    new.extend(carry[j] + _DS * _V for j in js)
                return tuple(new)

            init = []
            for j in range(nvec):
                bg, sl = jsl(j)
                init.append(ib.at[slot, bg, sl][...])
            lax.fori_loop(0, _DG, dgloop, tuple(init))

        start_in(0, 0)
        start_in(1, 1)

        wait_in(0)
        compute(0)
        start_out(0, 0)
        start_in(2, 2)

        @pl.loop(0, (_SPW - 1) // 3)
        def _(i):
            for sub in range(3):
                t = 3 * i + 1 + sub
                slot = (1 + sub) % 3
                nslot = sub
                wait_in(slot)
                compute(slot)
                start_out(t, slot)

                @pl.when(t + 2 < _SPW)
                def _():
                    wait_out(nslot)
                    start_in(t + 2, nslot)

        for slot in range(3):
            wait_out(slot)

    return k(x5, m3, t_flat)


@jax.jit
def kernel(x, minutes, table):
    x5 = (x.transpose(1, 2, 0)
           .reshape(_S, _DG, _DS, _BG, _BL)
           .transpose(0, 1, 3, 2, 4))
    m3 = (minutes.astype(jnp.int32).transpose(1, 0)
          .reshape(_SG, 8, _BG, _BL)
          .transpose(0, 2, 1, 3)
          .reshape(_SG, _BG, 8 * _BL))
    t_flat = table.T.reshape(_D * _V)
    o5 = _sc_embed_add(x5, m3, t_flat)
    return (o5.transpose(0, 1, 3, 2, 4)
              .reshape(_S, _D, _B)
              .transpose(2, 0, 1))
